# baseline (speedup 1.0000x reference)
import functools

import jax
import jax.numpy as jnp
from jax import lax
from jax.experimental import pallas as pl
from jax.experimental.pallas import tpu as pltpu
from jax.experimental.pallas import tpu_sc as plsc

VOCAB = 1000000
EMB = 64
BATCH = 16384

NC = 2
NS = 16
NW = NC * NS
BPW = BATCH // NW
CH = 128
NCH = BPW // CH
LANES = 16
NBLK = BPW // LANES

_mesh = plsc.VectorSubcoreMesh(core_axis_name="c", subcore_axis_name="s")


@functools.partial(
    pl.kernel,
    mesh=_mesh,
    compiler_params=pltpu.CompilerParams(
        needs_layout_passes=False, use_tc_tiling_on_sc=False),
    out_type=jax.ShapeDtypeStruct((BATCH,), jnp.float32),
    scratch_types=[
        pltpu.VMEM((NCH, CH), jnp.int32),
        pltpu.VMEM((NCH, CH), jnp.int32),
        pltpu.VMEM((BPW, EMB), jnp.float32),
        pltpu.VMEM((BPW, EMB), jnp.float32),
        pltpu.VMEM((BPW,), jnp.float32),
        pltpu.SemaphoreType.DMA,
    ],
)
def _skipgram_sc(u_idx_hbm, v_idx_hbm, u_w_hbm, v_w_hbm, out_hbm,
                 uidx_v, vidx_v, urows_v, vrows_v, out_v, sem):
    wid = lax.axis_index("s") * NC + lax.axis_index("c")
    base = wid * BPW

    for j in range(NCH):
        pltpu.sync_copy(u_idx_hbm.at[pl.ds(base + j * CH, CH)], uidx_v.at[j])
        pltpu.sync_copy(v_idx_hbm.at[pl.ds(base + j * CH, CH)], vidx_v.at[j])

    copies = []
    for j in range(NCH):
        copies.append(pltpu.async_copy(
            u_w_hbm.at[uidx_v.at[j]], urows_v.at[pl.ds(j * CH, CH)], sem))
        copies.append(pltpu.async_copy(
            v_w_hbm.at[vidx_v.at[j]], vrows_v.at[pl.ds(j * CH, CH)], sem))
    for c in copies:
        c.wait()

    lane_iota = lax.iota(jnp.int32, LANES)

    def blk_body(blk, carry):
        res = jnp.zeros((LANES,), jnp.float32)
        for l in range(LANES):
            r = blk * LANES + l
            acc = jnp.zeros((LANES,), jnp.float32)
            for k in range(EMB // LANES):
                u = urows_v[r, pl.ds(k * LANES, LANES)]
                v = vrows_v[r, pl.ds(k * LANES, LANES)]
                acc = acc + u * v
            s = jnp.sum(acc)
            res = jnp.where(lane_iota == l, s, res)
        out_v[pl.ds(blk * LANES, LANES)] = res
        return carry

    lax.fori_loop(0, NBLK, blk_body, 0)

    pltpu.sync_copy(out_v, out_hbm.at[pl.ds(base, BPW)])


def kernel(u_idxs, v_idxs, u_weight, v_weight):
    return _skipgram_sc(u_idxs.astype(jnp.int32), v_idxs.astype(jnp.int32),
                        u_weight, v_weight)
